# Initial kernel scaffold; baseline (speedup 1.0000x reference)
import jax
import jax.numpy as jnp
from jax.experimental import pallas as pl


def kernel(x, edge_index, W0, b0, Wrel1, Wroot1, b1, Wrel2, Wroot2, b2, Wout, bout):
    raise NotImplementedError("write your pallas kernel here")


# speedup vs baseline: 1.5273x; 1.5273x over previous
import jax
import jax.numpy as jnp
from jax.experimental import pallas as pl

_N = 100000
_BN = 5000
_NB = _N // _BN
_SOFTPLUS_BIAS = 0.5413248546129181
_HIGH = jax.lax.Precision.HIGHEST


def _bdot(a, b):
    return jnp.dot(a.astype(jnp.bfloat16), b.astype(jnp.bfloat16),
                   preferred_element_type=jnp.float32)


def _h0_body(x_ref, w_ref, b_ref, o_ref):
    o_ref[...] = _bdot(x_ref[...], w_ref[...]) + b_ref[...]


def _layer1_body(agg_ref, h0_ref, c_ref, wrel_ref, wroot_ref, b_ref,
                 agg2_ref, rows_ref):
    k = pl.program_id(0)
    z = (
        _bdot(agg_ref[...], wrel_ref[...])
        + b_ref[...]
        + _bdot(h0_ref[...], wroot_ref[...])
    )
    h1 = jnp.tanh(z)

    @pl.when(k == 0)
    def _():
        agg2_ref[...] = jnp.zeros_like(agg2_ref)
        rows_ref[...] = h1[1:9, :]

    contrib = jax.lax.dot_general(
        c_ref[...], h1, (((0,), (0,)), ((), ())),
        precision=_HIGH, preferred_element_type=jnp.float32)
    agg2_ref[...] += contrib


def _tail_body(agg2_ref, rows_ref, wrel_ref, wroot_ref, b_ref,
               w0e_ref, w1e_ref, bout_ref, o_ref):
    agg2 = agg2_ref[0:8, :]
    z2 = (
        _bdot(agg2, wrel_ref[...])
        + b_ref[...]
        + _bdot(rows_ref[...], wroot_ref[...])
    )
    h2 = jnp.tanh(z2)
    h2b = h2.astype(jnp.bfloat16).astype(jnp.float32)
    w0e = w0e_ref[...].astype(jnp.bfloat16).astype(jnp.float32)
    w1e = w1e_ref[...].astype(jnp.bfloat16).astype(jnp.float32)
    loc = jnp.sum(h2b * w0e, axis=1) + bout_ref[...][:, 0]
    raw = jnp.sum(h2b * w1e, axis=1) + bout_ref[...][:, 1]
    scale = jax.nn.softplus(raw + _SOFTPLUS_BIAS)
    o_ref[...] = jnp.stack([loc, scale], axis=1)


def kernel(x, edge_index, W0, b0, Wrel1, Wroot1, b1, Wrel2, Wroot2, b2,
           Wout, bout):
    src = edge_index[0]
    dst = edge_index[1]

    h0 = pl.pallas_call(
        _h0_body,
        grid=(_NB,),
        in_specs=[
            pl.BlockSpec((_BN, 11), lambda i: (i, 0)),
            pl.BlockSpec((11, 64), lambda i: (0, 0)),
            pl.BlockSpec((1, 64), lambda i: (0, 0)),
        ],
        out_specs=pl.BlockSpec((_BN, 64), lambda i: (i, 0)),
        out_shape=jax.ShapeDtypeStruct((_N, 64), jnp.float32),
    )(x, W0, b0.reshape(1, 64))

    agg1 = jnp.zeros((_N, 64), jnp.float32).at[dst].add(h0[src])

    seg = jnp.where((dst >= 1) & (dst <= 8), dst - 1, 8)
    c9 = jnp.zeros((_N, 9), jnp.float32).at[src, seg].add(1.0)

    agg2, rows = pl.pallas_call(
        _layer1_body,
        grid=(_NB,),
        in_specs=[
            pl.BlockSpec((_BN, 64), lambda i: (i, 0)),
            pl.BlockSpec((_BN, 64), lambda i: (i, 0)),
            pl.BlockSpec((_BN, 9), lambda i: (i, 0)),
            pl.BlockSpec((64, 64), lambda i: (0, 0)),
            pl.BlockSpec((64, 64), lambda i: (0, 0)),
            pl.BlockSpec((1, 64), lambda i: (0, 0)),
        ],
        out_specs=[
            pl.BlockSpec((9, 64), lambda i: (0, 0)),
            pl.BlockSpec((8, 64), lambda i: (0, 0)),
        ],
        out_shape=[
            jax.ShapeDtypeStruct((9, 64), jnp.float32),
            jax.ShapeDtypeStruct((8, 64), jnp.float32),
        ],
    )(agg1, h0, c9, Wrel1, Wroot1, b1.reshape(1, 64))

    res = pl.pallas_call(
        _tail_body,
        in_specs=[
            pl.BlockSpec((9, 64), lambda: (0, 0)),
            pl.BlockSpec((8, 64), lambda: (0, 0)),
            pl.BlockSpec((64, 64), lambda: (0, 0)),
            pl.BlockSpec((64, 64), lambda: (0, 0)),
            pl.BlockSpec((1, 64), lambda: (0, 0)),
            pl.BlockSpec((8, 64), lambda: (0, 0)),
            pl.BlockSpec((8, 64), lambda: (0, 0)),
            pl.BlockSpec((8, 2), lambda: (0, 0)),
        ],
        out_specs=pl.BlockSpec((8, 2), lambda: (0, 0)),
        out_shape=jax.ShapeDtypeStruct((8, 2), jnp.float32),
    )(agg2, rows, Wrel2, Wroot2, b2.reshape(1, 64),
      Wout[:, :, 0], Wout[:, :, 1], bout)

    return (res[:, 0], res[:, 1])
